# Initial kernel scaffold; baseline (speedup 1.0000x reference)
import jax
import jax.numpy as jnp
from jax.experimental import pallas as pl


def kernel(u_hyp, weight):
    raise NotImplementedError("write your pallas kernel here")


# speedup vs baseline: 1.4445x; 1.4445x over previous
import functools

import jax
import jax.numpy as jnp
from jax import lax
from jax.experimental import pallas as pl
from jax.experimental.pallas import tpu as pltpu
from jax.experimental.pallas import tpu_sc as plsc

_N_E = 8192
_D = 256
_ROWS = 4608
_BLK = 128
_NBLK = _ROWS // _BLK
_CLIP = 1.0 + 1e-7
_BETA = 0.25

_NC = 2
_NS = 16
_NW = _NC * _NS
_BPW = _ROWS // _NW
_DP = 384


def _tc_body(xs_ref, xt_ref, ws_ref, wt_ref, idx_ref, em_ref, sc_ref):
    i = pl.program_id(0)

    @pl.when(i == 0)
    def _init():
        em_ref[...] = jnp.zeros_like(em_ref)

    dot = lax.dot_general(
        xs_ref[...], ws_ref[...], (((1,), (1,)), ((), ())),
        preferred_element_type=jnp.float32)
    z = xt_ref[0] * wt_ref[...] - dot
    z = jnp.maximum(z, _CLIP)
    _H = _N_E // 2
    z_lo, z_hi = z[:, :_H], z[:, _H:]
    zmin_lo = jnp.min(z_lo, axis=1, keepdims=True)
    zmin_hi = jnp.min(z_hi, axis=1, keepdims=True)
    zmin = jnp.minimum(zmin_lo, zmin_hi)
    hidx = lax.broadcasted_iota(jnp.int32, (_BLK, _H), 1)
    cand_lo = jnp.min(jnp.where(z_lo == zmin_lo, hidx, _N_E), axis=1,
                      keepdims=True)
    cand_hi = jnp.min(jnp.where(z_hi == zmin_hi, hidx + _H, _N_E), axis=1,
                      keepdims=True)
    idxk = jnp.where(zmin_hi <= zmin_lo, cand_hi, cand_lo)
    idx_ref[0] = idxk
    jidx = lax.broadcasted_iota(jnp.int32, (_BLK, _N_E), 1)
    em_ref[...] += jnp.sum((jidx == idxk).astype(jnp.float32), axis=0,
                           keepdims=True)

    @pl.when(i == _NBLK - 1)
    def _fin():
        e = em_ref[...] * (1.0 / _ROWS)
        em_ref[...] = e
        s = jnp.sum(e * jnp.log(e + 1e-10))
        entv = jnp.broadcast_to(-s, (1, 128))
        pplv = jnp.exp(entv)
        lane = lax.broadcasted_iota(jnp.int32, (1, 128), 1)
        sc_ref[...] = jnp.where(lane == 1, entv,
                                jnp.where(lane == 2, pplv, 0.0))


def _tc_pass(xs, xt3, ws, wt):
    return pl.pallas_call(
        _tc_body,
        grid=(_NBLK,),
        in_specs=[
            pl.BlockSpec((_BLK, _D), lambda i: (i, 0)),
            pl.BlockSpec((1, _BLK, 1), lambda i: (i, 0, 0)),
            pl.BlockSpec((_N_E, _D), lambda i: (0, 0)),
            pl.BlockSpec((1, _N_E), lambda i: (0, 0)),
        ],
        out_specs=[
            pl.BlockSpec((1, _BLK, 1), lambda i: (i, 0, 0)),
            pl.BlockSpec((1, _N_E), lambda i: (0, 0)),
            pl.BlockSpec((1, 128), lambda i: (0, 0)),
        ],
        out_shape=[
            jax.ShapeDtypeStruct((_NBLK, _BLK, 1), jnp.int32),
            jax.ShapeDtypeStruct((1, _N_E), jnp.float32),
            jax.ShapeDtypeStruct((1, 128), jnp.float32),
        ],
    )(xs, xt3, ws, wt)


def _loss_body(x_ref, q_ref, out_ref):
    x = x_ref[...]
    q = q_ref[...]
    t = x[:, 0:1] * q[:, 0:1]
    s = jnp.sum(x * q, axis=1, keepdims=True)
    z = jnp.maximum(2.0 * t - s, _CLIP)
    d = jnp.log(z + jnp.sqrt((z - 1.0) * (z + 1.0)))
    out_ref[...] = jnp.broadcast_to(jnp.sum(d) * ((1.0 + _BETA) / _ROWS),
                                    (1, 128))


def _loss_pass(flat, zq):
    return pl.pallas_call(
        _loss_body,
        in_specs=[
            pl.BlockSpec((_ROWS, _D + 1), lambda: (0, 0)),
            pl.BlockSpec((_ROWS, _D + 1), lambda: (0, 0)),
        ],
        out_specs=pl.BlockSpec((1, 128), lambda: (0, 0)),
        out_shape=jax.ShapeDtypeStruct((1, 128), jnp.float32),
    )(flat, zq)


def _sc_gather(table, idx):
    mesh = plsc.VectorSubcoreMesh(core_axis_name="c", subcore_axis_name="s")

    @functools.partial(
        pl.kernel, mesh=mesh,
        out_type=jax.ShapeDtypeStruct((_ROWS, _DP), jnp.float32),
        scratch_types=[
            pltpu.VMEM((_BPW,), jnp.int32),
            pltpu.VMEM((_BPW, _DP), jnp.float32),
            pltpu.SemaphoreType.DMA,
        ],
    )
    def k(table_hbm, idx_hbm, out_hbm, idx_v, rows_v, sem):
        wid = lax.axis_index("s") * _NC + lax.axis_index("c")
        base = wid * _BPW
        pltpu.sync_copy(idx_hbm.at[pl.ds(base, _BPW)], idx_v)
        pltpu.async_copy(table_hbm.at[idx_v], rows_v, sem).wait()
        pltpu.sync_copy(rows_v, out_hbm.at[pl.ds(base, _BPW)])

    return k(table, idx)


def kernel(u_hyp, weight):
    flat = u_hyp.reshape(_ROWS, _D + 1)
    xs = flat[:, 1:]
    xt3 = flat[:, 0].reshape(_NBLK, _BLK, 1)
    ws = weight[:, 1:]
    wt = weight[:, 0].reshape(1, _N_E)
    idx3, em, scal = _tc_pass(xs, xt3, ws, wt)
    idx = idx3.reshape(_ROWS)
    wpad = jnp.pad(weight, ((0, 0), (0, _DP - (_D + 1))))
    zq_flat = _sc_gather(wpad, idx)[:, : _D + 1]
    z_q = zq_flat.reshape(u_hyp.shape)
    loss = _loss_pass(flat, zq_flat)[0, 0]
    diversity_loss = scal[0, 1]
    perplexity = scal[0, 2]
    codebook_usage = em.reshape(_N_E)
    return (loss, z_q, perplexity, diversity_loss, codebook_usage)


# speedup vs baseline: 1.5854x; 1.0975x over previous
import functools

import jax
import jax.numpy as jnp
from jax import lax
from jax.experimental import pallas as pl
from jax.experimental.pallas import tpu as pltpu
from jax.experimental.pallas import tpu_sc as plsc

_N_E = 8192
_D = 256
_ROWS = 4608
_BLK = 256
_NBLK = _ROWS // _BLK
_CLIP = 1.0 + 1e-7
_BETA = 0.25

_NC = 2
_NS = 16
_NW = _NC * _NS
_BPW = _ROWS // _NW
_NG = _BPW // 16
_DP = 384


def _tc_body(x_ref, w_ref, wt_ref, idx_ref, em_ref):
    i = pl.program_id(0)

    @pl.when(i == 0)
    def _init():
        em_ref[...] = jnp.zeros_like(em_ref)

    lanes = lax.broadcasted_iota(jnp.int32, (_BLK, _D + 1), 1)
    xm = jnp.where(lanes == 0, 0.0, x_ref[...])
    dot = lax.dot_general(
        xm, w_ref[...], (((1,), (1,)), ((), ())),
        preferred_element_type=jnp.float32)
    xt = x_ref[:, 0:1]
    z = xt * wt_ref[...] - dot
    z = jnp.maximum(z, _CLIP)
    _H = _N_E // 2
    z_lo, z_hi = z[:, :_H], z[:, _H:]
    zmin_lo = jnp.min(z_lo, axis=1, keepdims=True)
    zmin_hi = jnp.min(z_hi, axis=1, keepdims=True)
    hidx = lax.broadcasted_iota(jnp.int32, (_BLK, _H), 1)
    cand_lo = jnp.min(jnp.where(z_lo == zmin_lo, hidx, _N_E), axis=1,
                      keepdims=True)
    cand_hi = jnp.min(jnp.where(z_hi == zmin_hi, hidx + _H, _N_E), axis=1,
                      keepdims=True)
    idxk = jnp.where(zmin_hi <= zmin_lo, cand_hi, cand_lo)
    idx_ref[0] = idxk
    jidx = lax.broadcasted_iota(jnp.int32, (_BLK, _N_E), 1)
    em_ref[...] += jnp.sum((jidx == idxk).astype(jnp.float32), axis=0,
                           keepdims=True)


def _tc_pass(flat, weight, wt):
    return pl.pallas_call(
        _tc_body,
        grid=(_NBLK,),
        in_specs=[
            pl.BlockSpec((_BLK, _D + 1), lambda i: (i, 0)),
            pl.BlockSpec((_N_E, _D + 1), lambda i: (0, 0)),
            pl.BlockSpec((1, _N_E), lambda i: (0, 0)),
        ],
        out_specs=[
            pl.BlockSpec((1, _BLK, 1), lambda i: (i, 0, 0)),
            pl.BlockSpec((1, _N_E), lambda i: (0, 0)),
        ],
        out_shape=[
            jax.ShapeDtypeStruct((_NBLK, _BLK, 1), jnp.int32),
            jax.ShapeDtypeStruct((1, _N_E), jnp.float32),
        ],
    )(flat, weight, wt)


def _loss_body(x_ref, q_ref, cnt_ref, em_ref, sc_ref):
    x = x_ref[...]
    q = q_ref[...]
    t = x[:, 0:1] * q[:, 0:1]
    s = jnp.sum(x * q, axis=1, keepdims=True)
    z = jnp.maximum(2.0 * t - s, _CLIP)
    d = jnp.log(z + jnp.sqrt((z - 1.0) * (z + 1.0)))
    loss = jnp.sum(d) * ((1.0 + _BETA) / _ROWS)
    e = cnt_ref[...] * (1.0 / _ROWS)
    em_ref[...] = e
    ent = -jnp.sum(e * jnp.log(e + 1e-10))
    lossv = jnp.broadcast_to(loss, (1, 128))
    entv = jnp.broadcast_to(ent, (1, 128))
    pplv = jnp.exp(entv)
    lane = lax.broadcasted_iota(jnp.int32, (1, 128), 1)
    sc_ref[...] = jnp.where(lane == 0, lossv,
                            jnp.where(lane == 1, entv,
                                      jnp.where(lane == 2, pplv, 0.0)))


def _loss_pass(flat, zq, cnt):
    return pl.pallas_call(
        _loss_body,
        in_specs=[
            pl.BlockSpec((_ROWS, _D + 1), lambda: (0, 0)),
            pl.BlockSpec((_ROWS, _D + 1), lambda: (0, 0)),
            pl.BlockSpec((1, _N_E), lambda: (0, 0)),
        ],
        out_specs=[
            pl.BlockSpec((1, _N_E), lambda: (0, 0)),
            pl.BlockSpec((1, 128), lambda: (0, 0)),
        ],
        out_shape=[
            jax.ShapeDtypeStruct((1, _N_E), jnp.float32),
            jax.ShapeDtypeStruct((1, 128), jnp.float32),
        ],
    )(flat, zq, cnt)


def _sc_gather(table, idx):
    mesh = plsc.VectorSubcoreMesh(core_axis_name="c", subcore_axis_name="s")

    @functools.partial(
        pl.kernel, mesh=mesh,
        out_type=jax.ShapeDtypeStruct((_ROWS, _DP), jnp.float32),
        scratch_types=[
            pltpu.VMEM((_BPW,), jnp.int32),
            pltpu.VMEM((_BPW, _DP), jnp.float32),
            pltpu.SemaphoreType.DMA,
        ],
    )
    def k(table_hbm, idx_hbm, out_hbm, idx_v, rows_v, sem):
        wid = lax.axis_index("s") * _NC + lax.axis_index("c")
        base = wid * _BPW
        pltpu.sync_copy(idx_hbm.at[pl.ds(base, _BPW)], idx_v)
        pltpu.async_copy(table_hbm.at[idx_v], rows_v, sem).wait()
        pltpu.sync_copy(rows_v, out_hbm.at[pl.ds(base, _BPW)])

    return k(table, idx)


def kernel(u_hyp, weight):
    flat = u_hyp.reshape(_ROWS, _D + 1)
    wt = weight[:, 0].reshape(1, _N_E)
    idx3, cnt = _tc_pass(flat, weight, wt)
    idx = idx3.reshape(_ROWS)
    wpad = jnp.pad(weight, ((0, 0), (0, _DP - (_D + 1))))
    rows = _sc_gather(wpad, idx)
    zq_flat = rows[:, : _D + 1]
    z_q = zq_flat.reshape(u_hyp.shape)
    em, scal = _loss_pass(flat, zq_flat, cnt)
    loss = scal[0, 0]
    diversity_loss = scal[0, 1]
    perplexity = scal[0, 2]
    codebook_usage = em.reshape(_N_E)
    return (loss, z_q, perplexity, diversity_loss, codebook_usage)
